# Initial kernel scaffold; baseline (speedup 1.0000x reference)
import jax
import jax.numpy as jnp
from jax.experimental import pallas as pl


def kernel(x, edge_index, W, b):
    raise NotImplementedError("write your pallas kernel here")


# speedup vs baseline: 4.6999x; 4.6999x over previous
import functools

import jax
import jax.numpy as jnp
from jax import lax
from jax.experimental import pallas as pl
from jax.experimental.pallas import tpu as pltpu
from jax.experimental.pallas import tpu_sc as plsc

D = 128
_NC, _NS, _NL = 2, 16, 16
_NW = _NC * _NS
_CH = 80


def _mm_body(x_ref, w1_ref, w2_ref, b_ref, t1_ref, t2_ref):
    xb = x_ref[...]
    t1 = jnp.dot(xb, w1_ref[...], preferred_element_type=jnp.float32)
    t1_ref[...] = t1 + b_ref[0:1, :]
    t2_ref[...] = jnp.dot(xb, w2_ref[...], preferred_element_type=jnp.float32)


def _node_tables(x, W, b):
    n, d = x.shape
    blk = 1000
    t1, t2 = pl.pallas_call(
        _mm_body,
        grid=(n // blk,),
        in_specs=[
            pl.BlockSpec((blk, d), lambda i: (i, 0)),
            pl.BlockSpec((d, D), lambda i: (0, 0)),
            pl.BlockSpec((d, D), lambda i: (0, 0)),
            pl.BlockSpec((8, D), lambda i: (0, 0)),
        ],
        out_specs=[
            pl.BlockSpec((blk, D), lambda i: (i, 0)),
            pl.BlockSpec((blk, D), lambda i: (i, 0)),
        ],
        out_shape=[
            jax.ShapeDtypeStruct((n, D), jnp.float32),
            jax.ShapeDtypeStruct((n, D), jnp.float32),
        ],
    )(x, W[:d], W[d:], jnp.tile(b.reshape(1, D), (8, 1)))
    return t1, t2


def _edge_apply(t1, t2, src, dst):
    E = src.shape[0]
    epw = E // _NW
    nchunk = epw // _CH
    mesh = plsc.VectorSubcoreMesh(
        core_axis_name="c", subcore_axis_name="s",
        num_cores=_NC, num_subcores=_NS,
    )

    @functools.partial(
        pl.kernel,
        out_type=jax.ShapeDtypeStruct((E, D), jnp.float32),
        mesh=mesh,
        scratch_types=[
            pltpu.VMEM((epw,), jnp.int32),
            pltpu.VMEM((epw,), jnp.int32),
            pltpu.VMEM((_CH, D), jnp.float32),
            pltpu.VMEM((_CH, D), jnp.float32),
            pltpu.SemaphoreType.DMA,
            pltpu.SemaphoreType.DMA,
        ],
    )
    def k(t1_hbm, t2_hbm, src_hbm, dst_hbm, out_hbm,
          srcv, dstv, bufa, bufb, sema, semb):
        wid = lax.axis_index("s") * _NC + lax.axis_index("c")
        ebase = wid * epw
        pltpu.sync_copy(src_hbm.at[pl.ds(ebase, epw)], srcv)
        pltpu.sync_copy(dst_hbm.at[pl.ds(ebase, epw)], dstv)

        def chunk_body(c, carry):
            off = c * _CH
            cpa = pltpu.async_copy(
                t1_hbm.at[srcv.at[pl.ds(off, _CH)]], bufa, sema)
            cpb = pltpu.async_copy(
                t2_hbm.at[dstv.at[pl.ds(off, _CH)]], bufb, semb)
            cpa.wait()
            cpb.wait()

            def row_body(r, rcarry):
                for j in range(D // _NL):
                    sl = pl.ds(j * _NL, _NL)
                    bufa[r, sl] = jnp.maximum(bufa[r, sl] + bufb[r, sl], 0.0)
                return rcarry
            lax.fori_loop(0, _CH, row_body, 0)

            pltpu.sync_copy(bufa, out_hbm.at[pl.ds(ebase + off, _CH)])
            return carry

        lax.fori_loop(0, nchunk, chunk_body, 0)

    return k(t1, t2, src, dst)


def kernel(x, edge_index, W, b):
    t1, t2 = _node_tables(x, W, b)
    src = edge_index[0]
    dst = edge_index[1]
    return _edge_apply(t1, t2, src, dst)


# speedup vs baseline: 7.8280x; 1.6656x over previous
import functools

import jax
import jax.numpy as jnp
from jax import lax
from jax.experimental import pallas as pl
from jax.experimental.pallas import tpu as pltpu
from jax.experimental.pallas import tpu_sc as plsc

D = 128
_NC, _NS, _NL = 2, 16, 16
_NW = _NC * _NS
_CH = 80


def _mm_body(x_ref, w1_ref, w2_ref, b_ref, t1_ref, t2_ref):
    xb = x_ref[...]
    t1 = jnp.dot(xb, w1_ref[...], preferred_element_type=jnp.float32)
    t1_ref[...] = t1 + b_ref[0:1, :]
    t2_ref[...] = jnp.dot(xb, w2_ref[...], preferred_element_type=jnp.float32)


def _node_tables(x, W, b):
    n, d = x.shape
    blk = 1000
    t1, t2 = pl.pallas_call(
        _mm_body,
        grid=(n // blk,),
        in_specs=[
            pl.BlockSpec((blk, d), lambda i: (i, 0)),
            pl.BlockSpec((d, D), lambda i: (0, 0)),
            pl.BlockSpec((d, D), lambda i: (0, 0)),
            pl.BlockSpec((8, D), lambda i: (0, 0)),
        ],
        out_specs=[
            pl.BlockSpec((blk, D), lambda i: (i, 0)),
            pl.BlockSpec((blk, D), lambda i: (i, 0)),
        ],
        out_shape=[
            jax.ShapeDtypeStruct((n, D), jnp.float32),
            jax.ShapeDtypeStruct((n, D), jnp.float32),
        ],
    )(x, W[:d], W[d:], jnp.tile(b.reshape(1, D), (8, 1)))
    return t1, t2


def _edge_apply(t1, t2, src, dst):
    E = src.shape[0]
    epw = E // _NW
    nchunk = epw // _CH
    assert nchunk % 2 == 1 and nchunk >= 3
    npair = (nchunk - 1) // 2
    mesh = plsc.VectorSubcoreMesh(
        core_axis_name="c", subcore_axis_name="s",
        num_cores=_NC, num_subcores=_NS,
    )

    buf = pltpu.VMEM((_CH, D), jnp.float32)

    @functools.partial(
        pl.kernel,
        out_type=jax.ShapeDtypeStruct((E, D), jnp.float32),
        mesh=mesh,
        scratch_types=[
            pltpu.VMEM((epw,), jnp.int32),
            pltpu.VMEM((epw,), jnp.int32),
            buf, buf, buf, buf, buf, buf,
        ] + [pltpu.SemaphoreType.DMA] * 6,
    )
    def k(t1_hbm, t2_hbm, src_hbm, dst_hbm, out_hbm,
          srcv, dstv, ba0, bb0, bo0, ba1, bb1, bo1,
          sga0, sgb0, so0, sga1, sgb1, so1):
        wid = lax.axis_index("s") * _NC + lax.axis_index("c")
        ebase = wid * epw
        pltpu.sync_copy(src_hbm.at[pl.ds(ebase, epw)], srcv)
        pltpu.sync_copy(dst_hbm.at[pl.ds(ebase, epw)], dstv)

        bufs = ((ba0, bb0, bo0, sga0, sgb0, so0),
                (ba1, bb1, bo1, sga1, sgb1, so1))

        def gathers(c, s):
            ba, bb, _, sga, sgb, _ = bufs[s]
            off = c * _CH
            cpa = pltpu.make_async_copy(
                t1_hbm.at[srcv.at[pl.ds(off, _CH)]], ba, sga)
            cpb = pltpu.make_async_copy(
                t2_hbm.at[dstv.at[pl.ds(off, _CH)]], bb, sgb)
            return cpa, cpb

        def store_cp(c, s):
            _, _, bo, _, _, so = bufs[s]
            return pltpu.make_async_copy(
                bo, out_hbm.at[pl.ds(ebase + c * _CH, _CH)], so)

        def fire(c, s):
            cpa, cpb = gathers(c, s)
            cpa.start()
            cpb.start()

        def wait_gathers(c, s):
            cpa, cpb = gathers(c, s)
            cpa.wait()
            cpb.wait()

        def compute(s):
            ba, bb, bo, *_ = bufs[s]

            @plsc.parallel_loop(0, _CH, step=1, unroll=2)
            def _row(r):
                for j in range(D // _NL):
                    sl = pl.ds(j * _NL, _NL)
                    bo[r, sl] = jnp.maximum(ba[r, sl] + bb[r, sl], 0.0)

        fire(0, 0)
        fire(1, 1)

        def pair(p, carry):
            c0 = 2 * p
            wait_gathers(c0, 0)

            @pl.when(p > 0)
            def _():
                store_cp(c0 - 2, 0).wait()

            compute(0)
            store_cp(c0, 0).start()
            fire(c0 + 2, 0)

            c1 = c0 + 1
            wait_gathers(c1, 1)

            @pl.when(p > 0)
            def _():
                store_cp(c1 - 2, 1).wait()

            compute(1)
            store_cp(c1, 1).start()

            @pl.when(p < npair - 1)
            def _():
                fire(c1 + 2, 1)

            return carry

        lax.fori_loop(0, npair, pair, 0)

        clast = nchunk - 1
        wait_gathers(clast, 0)
        store_cp(clast - 2, 0).wait()
        compute(0)
        store_cp(clast, 0).start()
        store_cp(clast - 1, 1).wait()
        store_cp(clast, 0).wait()

    return k(t1, t2, src, dst)


def kernel(x, edge_index, W, b):
    t1, t2 = _node_tables(x, W, b)
    src = edge_index[0]
    dst = edge_index[1]
    return _edge_apply(t1, t2, src, dst)


# speedup vs baseline: 8.0794x; 1.0321x over previous
import functools

import jax
import jax.numpy as jnp
from jax import lax
from jax.experimental import pallas as pl
from jax.experimental.pallas import tpu as pltpu
from jax.experimental.pallas import tpu_sc as plsc

D = 128
_NC, _NS, _NL = 2, 16, 16
_NW = _NC * _NS
_CH = 80


def _mm_body(x_ref, w1_ref, w2_ref, b_ref, t1_ref, t2_ref):
    xb = x_ref[...]
    t1 = jnp.dot(xb, w1_ref[...], preferred_element_type=jnp.float32)
    t1_ref[...] = t1 + b_ref[0:1, :]
    t2_ref[...] = jnp.dot(xb, w2_ref[...], preferred_element_type=jnp.float32)


def _node_tables(x, W, b):
    n, d = x.shape
    blk = 1000
    t1, t2 = pl.pallas_call(
        _mm_body,
        grid=(n // blk,),
        in_specs=[
            pl.BlockSpec((blk, d), lambda i: (i, 0)),
            pl.BlockSpec((d, D), lambda i: (0, 0)),
            pl.BlockSpec((d, D), lambda i: (0, 0)),
            pl.BlockSpec((8, D), lambda i: (0, 0)),
        ],
        out_specs=[
            pl.BlockSpec((blk, D), lambda i: (i, 0)),
            pl.BlockSpec((blk, D), lambda i: (i, 0)),
        ],
        out_shape=[
            jax.ShapeDtypeStruct((n, D), jnp.float32),
            jax.ShapeDtypeStruct((n, D), jnp.float32),
        ],
    )(x, W[:d], W[d:], jnp.tile(b.reshape(1, D), (8, 1)))
    return t1, t2


def _edge_apply(t1, t2, src, dst):
    E = src.shape[0]
    epw = E // _NW
    nchunk = epw // _CH
    assert nchunk % 3 == 2 and nchunk >= 5
    ntri = (nchunk - 2) // 3
    mesh = plsc.VectorSubcoreMesh(
        core_axis_name="c", subcore_axis_name="s",
        num_cores=_NC, num_subcores=_NS,
    )

    buf = pltpu.VMEM((_CH, D), jnp.float32)

    @functools.partial(
        pl.kernel,
        out_type=jax.ShapeDtypeStruct((E, D), jnp.float32),
        mesh=mesh,
        scratch_types=[
            pltpu.VMEM((epw,), jnp.int32),
            pltpu.VMEM((epw,), jnp.int32),
            buf, buf, buf, buf, buf, buf, buf, buf, buf,
        ] + [pltpu.SemaphoreType.DMA] * 9,
    )
    def k(t1_hbm, t2_hbm, src_hbm, dst_hbm, out_hbm,
          srcv, dstv, ba0, bb0, bo0, ba1, bb1, bo1, ba2, bb2, bo2,
          sga0, sgb0, so0, sga1, sgb1, so1, sga2, sgb2, so2):
        wid = lax.axis_index("s") * _NC + lax.axis_index("c")
        ebase = wid * epw
        pltpu.sync_copy(src_hbm.at[pl.ds(ebase, epw)], srcv)
        pltpu.sync_copy(dst_hbm.at[pl.ds(ebase, epw)], dstv)

        bufs = ((ba0, bb0, bo0, sga0, sgb0, so0),
                (ba1, bb1, bo1, sga1, sgb1, so1),
                (ba2, bb2, bo2, sga2, sgb2, so2))

        def gathers(c, s):
            ba, bb, _, sga, sgb, _ = bufs[s]
            off = c * _CH
            cpa = pltpu.make_async_copy(
                t1_hbm.at[srcv.at[pl.ds(off, _CH)]], ba, sga)
            cpb = pltpu.make_async_copy(
                t2_hbm.at[dstv.at[pl.ds(off, _CH)]], bb, sgb)
            return cpa, cpb

        def store_cp(c, s):
            _, _, bo, _, _, so = bufs[s]
            return pltpu.make_async_copy(
                bo, out_hbm.at[pl.ds(ebase + c * _CH, _CH)], so)

        def fire(c, s):
            cpa, cpb = gathers(c, s)
            cpa.start()
            cpb.start()

        def wait_gathers(c, s):
            cpa, cpb = gathers(c, s)
            cpa.wait()
            cpb.wait()

        def compute(s):
            ba, bb, bo, *_ = bufs[s]

            @plsc.parallel_loop(0, _CH, step=1, unroll=2)
            def _row(r):
                for j in range(D // _NL):
                    sl = pl.ds(j * _NL, _NL)
                    bo[r, sl] = jnp.maximum(ba[r, sl] + bb[r, sl], 0.0)

        fire(0, 0)
        fire(1, 1)
        fire(2, 2)

        def tri(q, carry):
            for s in range(3):
                c = 3 * q + s
                wait_gathers(c, s)

                @pl.when(q > 0)
                def _():
                    store_cp(c - 3, s).wait()

                compute(s)
                store_cp(c, s).start()
                if s < 2:
                    fire(c + 3, s)
                else:
                    @pl.when(q < ntri - 1)
                    def _():
                        fire(c + 3, s)
            return carry

        lax.fori_loop(0, ntri, tri, 0)

        for s, c in ((0, nchunk - 2), (1, nchunk - 1)):
            wait_gathers(c, s)
            store_cp(c - 3, s).wait()
            compute(s)
            store_cp(c, s).start()
        store_cp(nchunk - 3, 2).wait()
        store_cp(nchunk - 2, 0).wait()
        store_cp(nchunk - 1, 1).wait()

    return k(t1, t2, src, dst)


def kernel(x, edge_index, W, b):
    t1, t2 = _node_tables(x, W, b)
    src = edge_index[0]
    dst = edge_index[1]
    return _edge_apply(t1, t2, src, dst)
